# baseline (speedup 1.0000x reference)
import functools

import jax
import jax.numpy as jnp
from jax.experimental import pallas as pl

_NUM_EMBEDDINGS = 8192
_EMBEDDING_DIM = 32
_COMMITMENT_COST = 0.25
_BLOCK_T = 256


def _vq_block(x_ref, emb_ref, embt_ref, out_ref, loss_ref):
    i = pl.program_id(0)
    x = x_ref[...]
    emb = emb_ref[...]
    embt = embt_ref[...]

    x2 = jnp.sum(x * x, axis=1, keepdims=True)
    e2 = jnp.sum(embt * embt, axis=0, keepdims=True)
    xe = jax.lax.dot_general(
        x, embt, (((1,), (0,)), ((), ())),
        precision=jax.lax.Precision.DEFAULT,
        preferred_element_type=jnp.float32)
    d = (x2 - 2.0 * xe) + e2

    d_min = jnp.min(d, axis=1, keepdims=True)
    iota = jax.lax.broadcasted_iota(jnp.int32, d.shape, 1)
    idx = jnp.min(jnp.where(d == d_min, iota, jnp.int32(2**30)),
                  axis=1, keepdims=True)
    one_hot = (iota == idx).astype(jnp.float32)
    q = jax.lax.dot_general(
        one_hot, emb, (((1,), (0,)), ((), ())),
        precision=jax.lax.Precision.HIGHEST,
        preferred_element_type=jnp.float32)

    diff = q - x
    out_ref[...] = x + diff

    @pl.when(i == 0)
    def _():
        loss_ref[...] = jnp.zeros((1, 1), jnp.float32)
    loss_ref[...] += jnp.sum(diff * diff, axis=(0, 1), keepdims=True)


def kernel(inputs, embeddings):
    x = inputs.reshape(-1, _EMBEDDING_DIM)
    n_tok = x.shape[0]
    grid = n_tok // _BLOCK_T
    embt = embeddings.T

    out, loss_sum = pl.pallas_call(
        _vq_block,
        grid=(grid,),
        in_specs=[
            pl.BlockSpec((_BLOCK_T, _EMBEDDING_DIM), lambda i: (i, 0)),
            pl.BlockSpec((_NUM_EMBEDDINGS, _EMBEDDING_DIM), lambda i: (0, 0)),
            pl.BlockSpec((_EMBEDDING_DIM, _NUM_EMBEDDINGS), lambda i: (0, 0)),
        ],
        out_specs=[
            pl.BlockSpec((_BLOCK_T, _EMBEDDING_DIM), lambda i: (i, 0)),
            pl.BlockSpec((1, 1), lambda i: (0, 0)),
        ],
        out_shape=[
            jax.ShapeDtypeStruct((n_tok, _EMBEDDING_DIM), jnp.float32),
            jax.ShapeDtypeStruct((1, 1), jnp.float32),
        ],
    )(x, embeddings, embt)

    quantized_st = out.reshape(inputs.shape)
    loss = (1.0 + _COMMITMENT_COST) * loss_sum[0, 0] / x.size
    return quantized_st, loss


# speedup vs baseline: 1.6512x; 1.6512x over previous
import jax
import jax.numpy as jnp
from jax.experimental import pallas as pl

_K = 8192
_D = 32
_COMMITMENT_COST = 0.25
_T = 256
_HALF = _K // 2


def _vq_block(x_ref, x2_ref, emb_hi_ref, emb_lo_ref, embt_ref, e2_ref,
              out_ref, loss_ref):
    i = pl.program_id(0)
    x = x_ref[...]
    x2 = x2_ref[...]
    embt = embt_ref[...]
    e2 = e2_ref[...]

    xe = jax.lax.dot_general(
        x, embt, (((1,), (0,)), ((), ())),
        precision=jax.lax.Precision.DEFAULT,
        preferred_element_type=jnp.float32)
    d = (x2 - 2.0 * xe) + e2

    iota = jax.lax.broadcasted_iota(jnp.int32, (_T, _K), 1)
    big = jnp.int32(2**30)

    d0 = d[:, :_HALF]
    d1 = d[:, _HALF:]
    v0 = jnp.min(d0, axis=1, keepdims=True)
    v1 = jnp.min(d1, axis=1, keepdims=True)
    i0 = jnp.min(jnp.where(d0 == v0, iota[:, :_HALF], big),
                 axis=1, keepdims=True)
    i1 = jnp.min(jnp.where(d1 == v1, iota[:, _HALF:], big),
                 axis=1, keepdims=True)
    v0_b = v0.astype(jnp.bfloat16).astype(jnp.float32)
    pick = jnp.where(v1 < v0_b, i1, i0)

    one_hot = (iota == pick).astype(jnp.float32)
    q_hi = jax.lax.dot_general(
        one_hot, emb_hi_ref[...], (((1,), (0,)), ((), ())),
        precision=jax.lax.Precision.DEFAULT,
        preferred_element_type=jnp.float32)
    q_lo = jax.lax.dot_general(
        one_hot, emb_lo_ref[...], (((1,), (0,)), ((), ())),
        precision=jax.lax.Precision.DEFAULT,
        preferred_element_type=jnp.float32)
    q = q_hi + q_lo

    diff = q - x
    out_ref[...] = x + diff

    @pl.when(i == 0)
    def _():
        loss_ref[...] = jnp.zeros((1, 1), jnp.float32)
    loss_ref[...] += jnp.sum(diff * diff, axis=(0, 1), keepdims=True)


def kernel(inputs, embeddings):
    x = inputs.reshape(-1, _D)
    n_tok = x.shape[0]
    grid = n_tok // _T

    x2 = jnp.sum(x ** 2, axis=1, keepdims=True)
    e2 = jnp.sum(embeddings ** 2, axis=1)[None, :]
    embt = embeddings.T
    emb_hi = embeddings.astype(jnp.bfloat16).astype(jnp.float32)
    emb_lo = embeddings - emb_hi

    out, loss_sum = pl.pallas_call(
        _vq_block,
        grid=(grid,),
        in_specs=[
            pl.BlockSpec((_T, _D), lambda i: (i, 0)),
            pl.BlockSpec((_T, 1), lambda i: (i, 0)),
            pl.BlockSpec((_K, _D), lambda i: (0, 0)),
            pl.BlockSpec((_K, _D), lambda i: (0, 0)),
            pl.BlockSpec((_D, _K), lambda i: (0, 0)),
            pl.BlockSpec((1, _K), lambda i: (0, 0)),
        ],
        out_specs=[
            pl.BlockSpec((_T, _D), lambda i: (i, 0)),
            pl.BlockSpec((1, 1), lambda i: (0, 0)),
        ],
        out_shape=[
            jax.ShapeDtypeStruct((n_tok, _D), jnp.float32),
            jax.ShapeDtypeStruct((1, 1), jnp.float32),
        ],
    )(x, x2, emb_hi, emb_lo, embt, e2)

    quantized_st = out.reshape(inputs.shape)
    loss = (1.0 + _COMMITMENT_COST) * loss_sum[0, 0] / x.size
    return quantized_st, loss


# speedup vs baseline: 1.7583x; 1.0649x over previous
import jax
import jax.numpy as jnp
from jax.experimental import pallas as pl
from jax.experimental.pallas import tpu as pltpu

_K = 8192
_D = 32
_COMMITMENT_COST = 0.25
_T = 512
_HALF = _K // 2


def _vq_block(x_ref, x2_ref, emb_hi_ref, emb_lo_ref, embt_ref, e2_ref,
              out_ref, loss_ref):
    x = x_ref[...]
    x2 = x2_ref[...]
    embt = embt_ref[...]
    e2 = e2_ref[...]

    xe = jax.lax.dot_general(
        x, embt, (((1,), (0,)), ((), ())),
        precision=jax.lax.Precision.DEFAULT,
        preferred_element_type=jnp.float32)
    d = (x2 - 2.0 * xe) + e2

    iota = jax.lax.broadcasted_iota(jnp.int32, (_T, _K), 1)
    big = jnp.int32(2**30)

    d0 = d[:, :_HALF]
    d1 = d[:, _HALF:]
    v0 = jnp.min(d0, axis=1, keepdims=True)
    v1 = jnp.min(d1, axis=1, keepdims=True)
    i0 = jnp.min(jnp.where(d0 == v0, iota[:, :_HALF], big),
                 axis=1, keepdims=True)
    i1 = jnp.min(jnp.where(d1 == v1, iota[:, _HALF:], big),
                 axis=1, keepdims=True)
    v0_b = v0.astype(jnp.bfloat16).astype(jnp.float32)
    pick = jnp.where(v1 < v0_b, i1, i0)

    one_hot = (iota == pick).astype(jnp.float32)
    q_hi = jax.lax.dot_general(
        one_hot, emb_hi_ref[...], (((1,), (0,)), ((), ())),
        precision=jax.lax.Precision.DEFAULT,
        preferred_element_type=jnp.float32)
    q_lo = jax.lax.dot_general(
        one_hot, emb_lo_ref[...], (((1,), (0,)), ((), ())),
        precision=jax.lax.Precision.DEFAULT,
        preferred_element_type=jnp.float32)
    q = q_hi + q_lo

    diff = q - x
    out_ref[...] = x + diff
    loss_ref[...] = jnp.sum(diff * diff, axis=(0, 1), keepdims=True)[None]


def kernel(inputs, embeddings):
    x = inputs.reshape(-1, _D)
    n_tok = x.shape[0]
    grid = n_tok // _T

    x2 = jnp.sum(x ** 2, axis=1, keepdims=True)
    e2 = jnp.sum(embeddings ** 2, axis=1)[None, :]
    embt = embeddings.T
    emb_hi = embeddings.astype(jnp.bfloat16).astype(jnp.float32)
    emb_lo = embeddings - emb_hi

    out, loss_sum = pl.pallas_call(
        _vq_block,
        grid=(grid,),
        in_specs=[
            pl.BlockSpec((_T, _D), lambda i: (i, 0)),
            pl.BlockSpec((_T, 1), lambda i: (i, 0)),
            pl.BlockSpec((_K, _D), lambda i: (0, 0)),
            pl.BlockSpec((_K, _D), lambda i: (0, 0)),
            pl.BlockSpec((_D, _K), lambda i: (0, 0)),
            pl.BlockSpec((1, _K), lambda i: (0, 0)),
        ],
        out_specs=[
            pl.BlockSpec((_T, _D), lambda i: (i, 0)),
            pl.BlockSpec((1, 1, 1), lambda i: (i, 0, 0)),
        ],
        out_shape=[
            jax.ShapeDtypeStruct((n_tok, _D), jnp.float32),
            jax.ShapeDtypeStruct((grid, 1, 1), jnp.float32),
        ],
        compiler_params=pltpu.CompilerParams(
            dimension_semantics=("parallel",)),
    )(x, x2, emb_hi, emb_lo, embt, e2)

    quantized_st = out.reshape(inputs.shape)
    loss = (1.0 + _COMMITMENT_COST) * jnp.sum(loss_sum) / x.size
    return quantized_st, loss
